# baseline (speedup 1.0000x reference)
import functools

import jax
import jax.numpy as jnp
from jax import lax
from jax.experimental import pallas as pl
from jax.experimental.pallas import tpu as pltpu
from jax.experimental.pallas import tpu_sc as plsc

D = 256
HALF = 128
N = 10000
E = 160000

NC = 2
NS = 16
CH = 128
NCH = 79
EP = NS * NCH * CH
EPT = NCH * CH
NP = 10240
RPT = NP // NS
BM = 1000


def _nf_body(x_ref, wn_ref, bn_ref, out_ref):
    nf = lax.dot_general(x_ref[...], wn_ref[...], (((1,), (1,)), ((), ())),
                         preferred_element_type=jnp.float32)
    nf = nf + bn_ref[...]
    out_ref[0] = nf[:, :HALF]
    out_ref[1] = nf[:, HALF:]


def _nf_call(x_flat, Wn, bn2):
    return pl.pallas_call(
        _nf_body,
        grid=(N // BM,),
        in_specs=[
            pl.BlockSpec((BM, D), lambda i: (i, 0)),
            pl.BlockSpec((D, D), lambda i: (0, 0)),
            pl.BlockSpec((1, D), lambda i: (0, 0)),
        ],
        out_specs=pl.BlockSpec((NC, BM, HALF), lambda i: (0, i, 0)),
        out_shape=jax.ShapeDtypeStruct((NC, N, HALF), jnp.float32),
    )(x_flat, Wn, bn2)


@functools.cache
def _make_sc_aggr():
    mesh = plsc.VectorSubcoreMesh(core_axis_name="c", subcore_axis_name="s")

    @functools.partial(
        pl.kernel,
        out_type=jax.ShapeDtypeStruct((NC, NP, HALF), jnp.float32),
        mesh=mesh,
        scratch_types=[
            pltpu.VMEM_SHARED((NP, HALF), jnp.float32),
            pltpu.VMEM((NCH, CH), jnp.int32),
            pltpu.VMEM((NCH, CH), jnp.int32),
            pltpu.VMEM((NCH, CH), jnp.float32),
            pltpu.VMEM((CH, HALF), jnp.float32),
            pltpu.SemaphoreType.DMA,
        ],
    )
    def sc_aggr(nf2, col3, row3, w3, out, aggr_sh, colv, rowv, wv, msg, sem):
        c = lax.axis_index("c")
        s = lax.axis_index("s")

        pltpu.sync_copy(col3.at[s], colv)
        pltpu.sync_copy(row3.at[s], rowv)
        pltpu.sync_copy(w3.at[s], wv)

        zeros16 = jnp.zeros((16,), jnp.float32)

        def zero_row(i, carry):
            for j in range(HALF // 16):
                msg[i, pl.ds(j * 16, 16)] = zeros16
            return carry

        lax.fori_loop(0, CH, zero_row, 0)
        for t in range(RPT // CH):
            pltpu.async_copy(msg, aggr_sh.at[pl.ds(s * RPT + t * CH, CH)],
                             sem)
        for t in range(RPT // CH):
            pltpu.make_async_copy(
                msg, aggr_sh.at[pl.ds(s * RPT + t * CH, CH)], sem).wait()
        plsc.subcore_barrier()

        def chunk(k, carry):
            pltpu.async_copy(nf2.at[c].at[colv.at[k]], msg, sem).wait()

            def edge16(t, inner):
                wvec = wv[k, pl.ds(t * 16, 16)]
                for l in range(16):
                    wval = wvec[l]
                    e = t * 16 + l
                    for j in range(HALF // 16):
                        sl = pl.ds(j * 16, 16)
                        msg[e, sl] = msg[e, sl] * wval
                return inner

            lax.fori_loop(0, CH // 16, edge16, 0)
            pltpu.sync_copy(msg, aggr_sh.at[rowv.at[k]], add=True)
            return carry

        lax.fori_loop(0, NCH, chunk, 0)
        plsc.subcore_barrier()

        pltpu.sync_copy(aggr_sh.at[pl.ds(s * RPT, RPT)],
                        out.at[c].at[pl.ds(s * RPT, RPT)])

    return sc_aggr


def _sc_aggr(nf2, col3, row3, w3):
    return _make_sc_aggr()(nf2, col3, row3, w3)


def _mlp_body(x_ref, a2_ref, ws_ref, wg1_ref, wg2_ref, bs_ref, bg1_ref,
              bg2_ref, out_ref):
    x_blk = x_ref[...]
    sf = lax.dot_general(x_blk, ws_ref[...], (((1,), (1,)), ((), ())),
                         preferred_element_type=jnp.float32) + bs_ref[...]
    aggr = jnp.concatenate([a2_ref[0], a2_ref[1]], axis=-1)
    h = jnp.concatenate([sf, aggr], axis=-1)
    g = lax.dot_general(h, wg1_ref[...], (((1,), (1,)), ((), ())),
                        preferred_element_type=jnp.float32) + bg1_ref[...]
    g = 0.5 * g * (1.0 + lax.erf(g * (2.0 ** -0.5)))
    out = lax.dot_general(g, wg2_ref[...], (((1,), (1,)), ((), ())),
                          preferred_element_type=jnp.float32) + bg2_ref[...]
    out_ref[...] = x_blk + out


def _mlp_call(x_flat, aggr2, Ws, Wg1, Wg2, bs2, bg12, bg22):
    return pl.pallas_call(
        _mlp_body,
        grid=(N // BM,),
        in_specs=[
            pl.BlockSpec((BM, D), lambda i: (i, 0)),
            pl.BlockSpec((NC, BM, HALF), lambda i: (0, i, 0)),
            pl.BlockSpec((D, D), lambda i: (0, 0)),
            pl.BlockSpec((D, 2 * D), lambda i: (0, 0)),
            pl.BlockSpec((D, D), lambda i: (0, 0)),
            pl.BlockSpec((1, D), lambda i: (0, 0)),
            pl.BlockSpec((1, D), lambda i: (0, 0)),
            pl.BlockSpec((1, D), lambda i: (0, 0)),
        ],
        out_specs=pl.BlockSpec((BM, D), lambda i: (i, 0)),
        out_shape=jax.ShapeDtypeStruct((N, D), jnp.float32),
    )(x_flat, aggr2, Ws, Wg1, Wg2, bs2, bg12, bg22)


def kernel(x, edge_index, edge_values, Ws, bs, Wn, bn, Wg1, bg1, Wg2, bg2):
    x_flat = x[0]
    pad = EP - E
    ei = edge_index.astype(jnp.int32)
    row = jnp.pad(ei[0], (0, pad)).reshape(NS, NCH, CH)
    col = jnp.pad(ei[1], (0, pad)).reshape(NS, NCH, CH)
    w3 = jnp.pad(edge_values.astype(jnp.float32), (0, pad)).reshape(
        NS, NCH, CH)

    nf2 = _nf_call(x_flat, Wn, bn.reshape(1, D))
    aggr2 = _sc_aggr(nf2, col, row, w3)
    out = _mlp_call(x_flat, aggr2, Ws, Wg1, Wg2, bs.reshape(1, D),
                    bg1.reshape(1, D), bg2.reshape(1, D))
    return out[None]
